# baseline (speedup 1.0000x reference)
import functools

import jax
import jax.numpy as jnp
from jax import lax
from jax.experimental import pallas as pl
from jax.experimental.pallas import tpu as pltpu
from jax.experimental.pallas import tpu_sc as plsc

VOCAB = 1000000
EMBED_DIM = 64
BATCH = 16384
HIST = 50

TOTAL = BATCH * HIST
NUM_CORES = 2
NUM_SUBCORES = 16
NW = NUM_CORES * NUM_SUBCORES
ROWS_PER_W = TOTAL // NW
CHUNK = 400
NCHUNK = ROWS_PER_W // CHUNK
NBUF = 2
NGROUP = NCHUNK // NBUF
assert NCHUNK % NBUF == 0


@functools.partial(
    pl.kernel,
    out_type=jax.ShapeDtypeStruct((TOTAL, EMBED_DIM), jnp.float32),
    mesh=plsc.VectorSubcoreMesh(core_axis_name="c", subcore_axis_name="s"),
    scratch_types=[
        [pltpu.VMEM((CHUNK,), jnp.int32) for _ in range(NBUF)],
        [pltpu.VMEM((CHUNK, EMBED_DIM), jnp.float32) for _ in range(NBUF)],
        [pltpu.SemaphoreType.DMA for _ in range(NBUF)],
        [pltpu.SemaphoreType.DMA for _ in range(NBUF)],
    ],
    compiler_params=pltpu.CompilerParams(use_tc_tiling_on_sc=False),
)
def _gather_kernel(table_hbm, idx_hbm, out_hbm, idx_bufs, row_bufs, gsems, ssems):
    wid = lax.axis_index("s") * NUM_CORES + lax.axis_index("c")
    base = wid * ROWS_PER_W

    for b in range(NBUF):
        off = base + b * CHUNK
        pltpu.sync_copy(idx_hbm.at[pl.ds(off, CHUNK)], idx_bufs[b])
        pltpu.async_copy(table_hbm.at[idx_bufs[b]], row_bufs[b], gsems[b])

    def group(gi, carry):
        for b in range(NBUF):
            off = base + (gi * NBUF + b) * CHUNK
            pltpu.make_async_copy(
                table_hbm.at[idx_bufs[b]], row_bufs[b], gsems[b]).wait()
            pltpu.async_copy(row_bufs[b], out_hbm.at[pl.ds(off, CHUNK)], ssems[b])

            @pl.when(gi < NGROUP - 1)
            def _():
                off2 = off + NBUF * CHUNK
                pltpu.sync_copy(idx_hbm.at[pl.ds(off2, CHUNK)], idx_bufs[b])
                pltpu.make_async_copy(
                    row_bufs[b], out_hbm.at[pl.ds(off, CHUNK)], ssems[b]).wait()
                pltpu.async_copy(table_hbm.at[idx_bufs[b]], row_bufs[b], gsems[b])

        return carry

    lax.fori_loop(0, NGROUP, group, 0)

    for b in range(NBUF):
        off = base + ((NGROUP - 1) * NBUF + b) * CHUNK
        pltpu.make_async_copy(
            row_bufs[b], out_hbm.at[pl.ds(off, CHUNK)], ssems[b]).wait()


def kernel(input_ids, embedding):
    flat_idx = input_ids.reshape(-1).astype(jnp.int32)
    out = _gather_kernel(embedding, flat_idx)
    return out.reshape(BATCH, HIST, EMBED_DIM)


# speedup vs baseline: 1.0003x; 1.0003x over previous
import functools

import jax
import jax.numpy as jnp
from jax import lax
from jax.experimental import pallas as pl
from jax.experimental.pallas import tpu as pltpu
from jax.experimental.pallas import tpu_sc as plsc

VOCAB = 1000000
EMBED_DIM = 64
BATCH = 16384
HIST = 50

TOTAL = BATCH * HIST
NUM_CORES = 2
NUM_SUBCORES = 16
NW = NUM_CORES * NUM_SUBCORES
ROWS_PER_W = TOTAL // NW
CHUNK = 400
NCHUNK = ROWS_PER_W // CHUNK
NBUF = 4
NGROUP = NCHUNK // NBUF
assert NCHUNK % NBUF == 0


@functools.partial(
    pl.kernel,
    out_type=jax.ShapeDtypeStruct((TOTAL, EMBED_DIM), jnp.float32),
    mesh=plsc.VectorSubcoreMesh(core_axis_name="c", subcore_axis_name="s"),
    scratch_types=[
        [pltpu.VMEM((CHUNK,), jnp.int32) for _ in range(NBUF)],
        [pltpu.VMEM((CHUNK, EMBED_DIM), jnp.float32) for _ in range(NBUF)],
        [pltpu.SemaphoreType.DMA for _ in range(NBUF)],
        [pltpu.SemaphoreType.DMA for _ in range(NBUF)],
    ],
    compiler_params=pltpu.CompilerParams(use_tc_tiling_on_sc=False),
)
def _gather_kernel(table_hbm, idx_hbm, out_hbm, idx_bufs, row_bufs, gsems, ssems):
    wid = lax.axis_index("s") * NUM_CORES + lax.axis_index("c")
    base = wid * ROWS_PER_W

    for b in range(NBUF):
        off = base + b * CHUNK
        pltpu.sync_copy(idx_hbm.at[pl.ds(off, CHUNK)], idx_bufs[b])
        pltpu.async_copy(table_hbm.at[idx_bufs[b]], row_bufs[b], gsems[b])

    def group(gi, carry):
        for b in range(NBUF):
            off = base + (gi * NBUF + b) * CHUNK
            pltpu.make_async_copy(
                table_hbm.at[idx_bufs[b]], row_bufs[b], gsems[b]).wait()
            pltpu.async_copy(row_bufs[b], out_hbm.at[pl.ds(off, CHUNK)], ssems[b])

            @pl.when(gi < NGROUP - 1)
            def _():
                off2 = off + NBUF * CHUNK
                pltpu.sync_copy(idx_hbm.at[pl.ds(off2, CHUNK)], idx_bufs[b])
                pltpu.make_async_copy(
                    row_bufs[b], out_hbm.at[pl.ds(off, CHUNK)], ssems[b]).wait()
                pltpu.async_copy(table_hbm.at[idx_bufs[b]], row_bufs[b], gsems[b])

        return carry

    lax.fori_loop(0, NGROUP, group, 0)

    for b in range(NBUF):
        off = base + ((NGROUP - 1) * NBUF + b) * CHUNK
        pltpu.make_async_copy(
            row_bufs[b], out_hbm.at[pl.ds(off, CHUNK)], ssems[b]).wait()


def kernel(input_ids, embedding):
    flat_idx = input_ids.reshape(-1).astype(jnp.int32)
    out = _gather_kernel(embedding, flat_idx)
    return out.reshape(BATCH, HIST, EMBED_DIM)


# speedup vs baseline: 1.0538x; 1.0534x over previous
import functools

import jax
import jax.numpy as jnp
from jax import lax
from jax.experimental import pallas as pl
from jax.experimental.pallas import tpu as pltpu
from jax.experimental.pallas import tpu_sc as plsc

VOCAB = 1000000
EMBED_DIM = 64
BATCH = 16384
HIST = 50

TOTAL = BATCH * HIST
NUM_CORES = 2
NUM_SUBCORES = 16
NW = NUM_CORES * NUM_SUBCORES
ROWS_PER_W = TOTAL // NW
CHUNK = 400
NCHUNK = ROWS_PER_W // CHUNK
NBUF = 4
NGROUP = NCHUNK // NBUF
assert NCHUNK % NBUF == 0


@functools.partial(
    pl.kernel,
    out_type=jax.ShapeDtypeStruct((TOTAL, EMBED_DIM), jnp.float32),
    mesh=plsc.VectorSubcoreMesh(core_axis_name="c", subcore_axis_name="s"),
    scratch_types=[
        [pltpu.VMEM((CHUNK,), jnp.int32) for _ in range(NBUF)],
        [pltpu.VMEM((CHUNK, EMBED_DIM), jnp.float32) for _ in range(NBUF)],
        [pltpu.SemaphoreType.DMA for _ in range(NBUF)],
        [pltpu.SemaphoreType.DMA for _ in range(NBUF)],
    ],
    compiler_params=pltpu.CompilerParams(use_tc_tiling_on_sc=False),
)
def _gather_kernel(table_hbm, idx_hbm, out_hbm, idx_bufs, row_bufs, gsems, ssems):
    wid = lax.axis_index("s") * NUM_CORES + lax.axis_index("c")
    base = wid * ROWS_PER_W

    for b in range(NBUF):
        off = base + b * CHUNK
        pltpu.sync_copy(idx_hbm.at[pl.ds(off, CHUNK)], idx_bufs[b])
        pltpu.async_copy(table_hbm.at[idx_bufs[b]], row_bufs[b], gsems[b])

    def group(gi, carry):
        for b in range(NBUF):
            off = base + (gi * NBUF + b) * CHUNK
            pltpu.make_async_copy(
                table_hbm.at[idx_bufs[b]], row_bufs[b], gsems[b]).wait()
            pltpu.async_copy(row_bufs[b], out_hbm.at[pl.ds(off, CHUNK)], ssems[b])

            @pl.when(gi < NGROUP - 1)
            def _():
                off2 = off + NBUF * CHUNK
                pltpu.sync_copy(idx_hbm.at[pl.ds(off2, CHUNK)], idx_bufs[b])
                pltpu.make_async_copy(
                    row_bufs[b], out_hbm.at[pl.ds(off, CHUNK)], ssems[b]).wait()
                pltpu.async_copy(table_hbm.at[idx_bufs[b]], row_bufs[b], gsems[b])

        return carry

    lax.fori_loop(0, NGROUP, group, 0)

    for b in range(NBUF):
        off = base + ((NGROUP - 1) * NBUF + b) * CHUNK
        pltpu.make_async_copy(
            row_bufs[b], out_hbm.at[pl.ds(off, CHUNK)], ssems[b]).wait()


def kernel(input_ids, embedding):
    table_p = jnp.pad(embedding, ((0, 0), (0, 64))).reshape(2 * VOCAB, EMBED_DIM)
    flat_idx = (input_ids.reshape(-1) * 2).astype(jnp.int32)
    out = _gather_kernel(table_p, flat_idx)
    return out.reshape(BATCH, HIST, EMBED_DIM)
